# baseline (speedup 1.0000x reference)
import jax
import jax.numpy as jnp
from jax import lax
from jax.experimental import pallas as pl
from jax.experimental.pallas import tpu as pltpu
from jax.experimental.pallas import tpu_sc as plsc

_B = 16384
_D = 32
_NC = 2
_NS = 16
_NW = _NC * _NS
_BPW = _B // _NW
_SSW = 512
_ROWS_PAD = _B + _NW
_LAMBDA = 0.01

_LOG_COEFFS = (
    7.0376836292e-2, -1.1514610310e-1, 1.1676998740e-1, -1.2420140846e-1,
    1.4249322787e-1, -1.6668057665e-1, 2.0000714765e-1, -2.4999993993e-1,
    3.3333331174e-1,
)


def _logf(y):
    bits = lax.bitcast_convert_type(y, jnp.int32)
    e = lax.shift_right_logical(bits, 23) - 126
    m = lax.bitcast_convert_type(
        jnp.bitwise_or(jnp.bitwise_and(bits, 0x007FFFFF), 0x3F000000),
        jnp.float32)
    big = m > jnp.float32(0.70710678)
    e = jnp.where(big, e, e - 1).astype(jnp.float32)
    x = jnp.where(big, m - 1.0, m + m - 1.0)
    z = x * x
    p = jnp.full_like(x, _LOG_COEFFS[0])
    for c in _LOG_COEFFS[1:]:
        p = p * x + c
    r = x * z * p
    r = r + e * jnp.float32(-2.12194440e-4)
    r = r - jnp.float32(0.5) * z
    return x + r + e * jnp.float32(0.693359375)


def _log_sigmoid(x):
    t = jnp.exp(-jnp.abs(x))
    return jnp.minimum(x, 0.0) - _logf(1.0 + t)


def _scan_pass(tbl_h, tail_h, n_rows, idx_sets, out_h, wid,
               hb_v, hr_v, ring_v, idxc_v, stage_v, drain_v, sem, sem2):
    full = n_rows // _SSW
    tw = n_rows - full * _SSW
    iota16 = lax.iota(jnp.int32, 16)
    dump = jnp.full((16,), _B + wid, jnp.int32)

    nh_vec = jnp.zeros((16,), jnp.int32)
    for set_id, idx_h in enumerate(idx_sets):
        def blk_body(blk, nh_vec, idx_h=idx_h, set_id=set_id):
            pltpu.sync_copy(idx_h.at[blk], idxc_v)
            for q in range(8):
                v = idxc_v[pl.ds(q * 16, 16)]
                m = jnp.bitwise_and(
                    lax.shift_right_logical(v, 9), _NW - 1) == wid
                pos = nh_vec + plsc.cumsum(m.astype(jnp.int32)) - 1
                b_enc = blk * 128 + (q * 16 + set_id * _B) + iota16
                plsc.store_scatter(hb_v, [pos], b_enc, mask=m)
                plsc.store_scatter(hr_v, [pos], v, mask=m)
                nh_vec = nh_vec + plsc.all_reduce_population_count(m)
            return nh_vec
        nh_vec = lax.fori_loop(0, 128, blk_body, nh_vec)
    plsc.store_scatter(hr_v, [nh_vec + iota16],
                       jnp.full((16,), -1, jnp.int32))
    nhv = lax.shift_right_logical(jnp.max(nh_vec) + 15, 4)

    def match_hits(t, slot):
        def hit_body(h, carry2):
            off = pl.multiple_of(h * 16, 16)
            hbv = hb_v[pl.ds(off, 16)]
            hrv = hr_v[pl.ds(off, 16)]
            m = lax.shift_right_logical(hrv, 9) == t

            @pl.when(jnp.any(m))
            def _():
                pltpu.make_async_copy(out_h.at[pl.ds(_B, 16)],
                                      drain_v, sem2).wait()
                p = jnp.bitwise_and(hrv, _SSW - 1)
                for c in range(_D):
                    val = plsc.load_gather(
                        ring_v.at[slot], [jnp.full((16,), c, jnp.int32), p])
                    plsc.store_scatter(
                        stage_v, [iota16, jnp.full((16,), c, jnp.int32)],
                        val, mask=m)
                rows = (jnp.bitwise_and(hbv, _B - 1)
                        + lax.shift_right_logical(hbv, 14) * _ROWS_PAD)
                rows = jnp.where(m, rows, dump)
                pltpu.async_copy(stage_v, out_h.at[rows], sem2)
            return carry2

        lax.fori_loop(0, nhv, hit_body, 0)

    def issue(t, slot):
        @pl.when(t < full)
        def _():
            base = pl.multiple_of(t * _SSW, _SSW)
            pltpu.async_copy(tbl_h.at[:, pl.ds(base, _SSW)],
                             ring_v.at[slot], sem)

    for k in range(3):
        issue(wid + k * _NW, k)

    nt = (full - wid + _NW - 1) // _NW

    def slab_body(k, carry):
        t = wid + k * _NW
        slot = k % 3
        pltpu.make_async_copy(tbl_h.at[:, pl.ds(0, _SSW)],
                              ring_v.at[slot], sem).wait()
        match_hits(t, slot)
        issue(wid + (k + 3) * _NW, slot)
        return carry

    lax.fori_loop(0, nt, slab_body, 0)

    if tw:
        twp = tail_h.shape[1]

        @pl.when(wid == full % _NW)
        def _():
            pltpu.sync_copy(tail_h, ring_v.at[0, :, pl.ds(0, twp)])
            match_hits(full, 0)


def _ka_body(wt_h, ht_h, h1t_h, wtail_h, htail_h, h1tail_h,
             u_h, i_h, j_h, i1_h,
             ue_h, ieje_h, a_h,
             hb_v, hr_v, ring_v, idxc_v, stage_v, drain_v, sem, sem2):
    wid = lax.axis_index("s") * _NC + lax.axis_index("c")
    dumprows = jnp.full((16,), _B + wid, jnp.int32)
    pltpu.async_copy(stage_v, ue_h.at[dumprows], sem2)
    common = (wid, hb_v, hr_v, ring_v, idxc_v, stage_v, drain_v, sem, sem2)
    _scan_pass(wt_h, wtail_h, 1000000, [u_h], ue_h, *common)
    _scan_pass(ht_h, htail_h, 1000000, [i_h, j_h], ieje_h, *common)
    _scan_pass(h1t_h, h1tail_h, 100000, [i1_h], a_h, *common)
    pltpu.make_async_copy(ue_h.at[pl.ds(_B, 16)], drain_v, sem2).wait()


def _kb_body(ue_h, ieje_h, a_h, h2t_h, i2_h, r1_h, r2_h, out_h,
             ue_v, ie_v, je_v, a_v, h2_v, i2_v, r1_v, r2_v, p_v, sem):
    wid = lax.axis_index("s") * _NC + lax.axis_index("c")
    pltpu.sync_copy(h2t_h, h2_v)
    for k in range(4):
        pltpu.sync_copy(i2_h.at[wid * 4 + k], i2_v.at[pl.ds(k * 128, 128)])
        pltpu.sync_copy(r1_h.at[wid * 4 + k], r1_v.at[pl.ds(k * 128, 128)])
        pltpu.sync_copy(r2_h.at[wid * 4 + k], r2_v.at[pl.ds(k * 128, 128)])

    iota16 = lax.iota(jnp.int32, 16)
    zero = jnp.zeros((16,), jnp.float32)
    acc_total = zero

    for s in range(4):
        base = pl.multiple_of(wid * _BPW + s * 128, 128)
        copies = [
            pltpu.async_copy(ue_h.at[pl.ds(base, 128)], ue_v, sem),
            pltpu.async_copy(ieje_h.at[pl.ds(base, 128)], ie_v, sem),
            pltpu.async_copy(ieje_h.at[pl.ds(_ROWS_PAD + base, 128)],
                             je_v, sem),
            pltpu.async_copy(a_h.at[pl.ds(base, 128)], a_v, sem),
        ]
        for cp in copies:
            cp.wait()

        def block(b, acc, s=s):
            roff = pl.multiple_of(b * 16, 16)
            rvec = iota16 + roff
            goff = pl.multiple_of(s * 128 + b * 16, 16)
            i2vals = i2_v[pl.ds(goff, 16)]
            x = x1 = x2 = ru = q1 = q2 = zero
            for c in range(_D):
                cv = jnp.full((16,), c, jnp.int32)
                ue = plsc.load_gather(ue_v, [rvec, cv])
                ie = plsc.load_gather(ie_v, [rvec, cv])
                je = plsc.load_gather(je_v, [rvec, cv])
                ae = plsc.load_gather(a_v, [rvec, cv])
                ge = plsc.load_gather(h2_v, [cv, i2vals])
                x = x + ue * (ie - je)
                x1 = x1 + ue * ae
                x2 = x2 + ue * ge
                ru = ru + ue * ue
                d1 = ae - ie
                q1 = q1 + d1 * d1
                d2 = ge - ae
                q2 = q2 + d2 * d2
            ls = _log_sigmoid(x)
            t1 = r1_v[pl.ds(goff, 16)] - x1
            t2 = r2_v[pl.ds(goff, 16)] - x2
            return acc + (-ls + t1 * t1 + t2 * t2
                          + jnp.float32(_LAMBDA) * (ru + q1 + q2))

        acc_total = lax.fori_loop(0, 8, block, acc_total)

    p_v[...] = acc_total
    pltpu.sync_copy(p_v, out_h.at[wid])


def _tc_finish(p_ref, o_ref):
    o_ref[0, 0] = jnp.sum(p_ref[...])


@jax.jit
def _run(u2, i2, j2, i1_2, i2_2, W, H, H_1, H_2, r1_2, r2_2):
    mesh = plsc.VectorSubcoreMesh(core_axis_name="c", subcore_axis_name="s")
    params = pltpu.CompilerParams(needs_layout_passes=False)
    ka = pl.kernel(
        _ka_body,
        out_type=(
            jax.ShapeDtypeStruct((_ROWS_PAD, 128), jnp.float32),
            jax.ShapeDtypeStruct((2 * _ROWS_PAD, 128), jnp.float32),
            jax.ShapeDtypeStruct((_ROWS_PAD, 128), jnp.float32),
        ),
        mesh=mesh,
        compiler_params=params,
        scratch_types=[
            pltpu.VMEM((2 * _B + 32,), jnp.int32),
            pltpu.VMEM((2 * _B + 32,), jnp.int32),
            pltpu.VMEM((3, 32, _SSW), jnp.float32),
            pltpu.VMEM((128,), jnp.int32),
            pltpu.VMEM((16, 128), jnp.float32),
            pltpu.VMEM((16, 128), jnp.float32),
            pltpu.SemaphoreType.DMA,
            pltpu.SemaphoreType.DMA,
        ],
    )
    wtail = jnp.pad(W.T[:, 999936:], ((0, 0), (0, 64)))
    htail = jnp.pad(H.T[:, 999936:], ((0, 0), (0, 64)))
    h1tail = jnp.pad(H_1.T[:, 99840:], ((0, 0), (0, 96)))
    ue_buf, ieje_buf, a_buf = ka(W.T, H.T, H_1.T, wtail, htail, h1tail,
                                 u2, i2, j2, i1_2)

    kb = pl.kernel(
        _kb_body,
        out_type=jax.ShapeDtypeStruct((_NW, 16), jnp.float32),
        mesh=mesh,
        compiler_params=params,
        scratch_types=[
            pltpu.VMEM((128, 128), jnp.float32),
            pltpu.VMEM((128, 128), jnp.float32),
            pltpu.VMEM((128, 128), jnp.float32),
            pltpu.VMEM((128, 128), jnp.float32),
            pltpu.VMEM((32, 1000), jnp.float32),
            pltpu.VMEM((_BPW,), jnp.int32),
            pltpu.VMEM((_BPW,), jnp.float32),
            pltpu.VMEM((_BPW,), jnp.float32),
            pltpu.VMEM((16,), jnp.float32),
            pltpu.SemaphoreType.DMA,
        ],
    )
    partials = kb(ue_buf, ieje_buf, a_buf, H_2.T, i2_2, r1_2, r2_2)

    total = pl.pallas_call(
        _tc_finish,
        out_shape=jax.ShapeDtypeStruct((1, 1), jnp.float32),
        out_specs=pl.BlockSpec(memory_space=pltpu.SMEM),
    )(partials)
    return total[0, 0]


def kernel(u, i, j, i_1, i_2, batch_size, W, H, H_1, H_2, r_1, r_2):
    u2 = u.astype(jnp.int32).reshape(128, 128)
    i2 = i.astype(jnp.int32).reshape(128, 128)
    j2 = j.astype(jnp.int32).reshape(128, 128)
    i1_2 = i_1.astype(jnp.int32).reshape(128, 128)
    i2_2 = i_2.astype(jnp.int32).reshape(128, 128)
    r1_2 = r_1.reshape(128, 128)
    r2_2 = r_2.reshape(128, 128)
    total = _run(u2, i2, j2, i1_2, i2_2, W, H, H_1, H_2, r1_2, r2_2)
    return total / batch_size


# speedup vs baseline: 1.0348x; 1.0348x over previous
import jax
import jax.numpy as jnp
from jax import lax
from jax.experimental import pallas as pl
from jax.experimental.pallas import tpu as pltpu
from jax.experimental.pallas import tpu_sc as plsc

_B = 16384
_D = 32
_NC = 2
_NS = 16
_NW = _NC * _NS
_BPW = _B // _NW
_SSW = 512
_ROWS_PAD = _B + _NW
_LAMBDA = 0.01

_LOG_COEFFS = (
    7.0376836292e-2, -1.1514610310e-1, 1.1676998740e-1, -1.2420140846e-1,
    1.4249322787e-1, -1.6668057665e-1, 2.0000714765e-1, -2.4999993993e-1,
    3.3333331174e-1,
)


def _logf(y):
    bits = lax.bitcast_convert_type(y, jnp.int32)
    e = lax.shift_right_logical(bits, 23) - 126
    m = lax.bitcast_convert_type(
        jnp.bitwise_or(jnp.bitwise_and(bits, 0x007FFFFF), 0x3F000000),
        jnp.float32)
    big = m > jnp.float32(0.70710678)
    e = jnp.where(big, e, e - 1).astype(jnp.float32)
    x = jnp.where(big, m - 1.0, m + m - 1.0)
    z = x * x
    p = jnp.full_like(x, _LOG_COEFFS[0])
    for c in _LOG_COEFFS[1:]:
        p = p * x + c
    r = x * z * p
    r = r + e * jnp.float32(-2.12194440e-4)
    r = r - jnp.float32(0.5) * z
    return x + r + e * jnp.float32(0.693359375)


def _log_sigmoid(x):
    t = jnp.exp(-jnp.abs(x))
    return jnp.minimum(x, 0.0) - _logf(1.0 + t)


def _scan_pass(tbl_h, tail_h, n_rows, idx_sets, out_h, wid,
               hb_v, hr_v, ring_v, idxc_v, stage_v, drain_v, cnt_s,
               sem, sem2):
    full = n_rows // _SSW
    tw = n_rows - full * _SSW
    iota16 = lax.iota(jnp.int32, 16)
    dump = jnp.full((16,), _B + wid, jnp.int32)

    nh_vec = jnp.zeros((16,), jnp.int32)
    for set_id, idx_h in enumerate(idx_sets):
        pltpu.async_copy(idx_h.at[0], idxc_v.at[0], sem)

        def blk_body(blk, nh_vec, idx_h=idx_h, set_id=set_id):
            slot = blk % 2
            pltpu.make_async_copy(idx_h.at[0], idxc_v.at[slot], sem).wait()

            @pl.when(blk < 127)
            def _():
                pltpu.async_copy(idx_h.at[blk + 1],
                                 idxc_v.at[1 - slot], sem)
            for q in range(8):
                v = idxc_v[slot, pl.ds(q * 16, 16)]
                m = jnp.bitwise_and(
                    lax.shift_right_logical(v, 9), _NW - 1) == wid
                pos = nh_vec + plsc.cumsum(m.astype(jnp.int32)) - 1
                b_enc = blk * 128 + (q * 16 + set_id * _B) + iota16
                plsc.store_scatter(hb_v, [pos], b_enc, mask=m)
                plsc.store_scatter(hr_v, [pos], v, mask=m)
                nh_vec = nh_vec + plsc.all_reduce_population_count(m)
            return nh_vec
        nh_vec = lax.fori_loop(0, 128, blk_body, nh_vec)
    plsc.store_scatter(hr_v, [nh_vec + iota16],
                       jnp.full((16,), -1, jnp.int32))
    nhv = lax.shift_right_logical(jnp.max(nh_vec) + 15, 4)

    def match_hits(t, slot):
        def hit_body(h, carry2):
            off = pl.multiple_of(h * 16, 16)
            hbv = hb_v[pl.ds(off, 16)]
            hrv = hr_v[pl.ds(off, 16)]
            m = lax.shift_right_logical(hrv, 9) == t

            @pl.when(jnp.any(m))
            def _():
                cnt = cnt_s[0]
                sslot = lax.rem(cnt, 5)
                pltpu.make_async_copy(out_h.at[pl.ds(_B, 16)],
                                      drain_v, sem2).wait()
                p = jnp.bitwise_and(hrv, _SSW - 1)
                sv = jnp.full((16,), sslot, jnp.int32)
                for c in range(_D):
                    val = plsc.load_gather(
                        ring_v.at[slot], [jnp.full((16,), c, jnp.int32), p])
                    plsc.store_scatter(
                        stage_v,
                        [sv, iota16, jnp.full((16,), c, jnp.int32)],
                        val, mask=m)
                rows = (jnp.bitwise_and(hbv, _B - 1)
                        + lax.shift_right_logical(hbv, 14) * _ROWS_PAD)
                rows = jnp.where(m, rows, dump)
                pltpu.async_copy(stage_v.at[sslot], out_h.at[rows], sem2)
                cnt_s[0] = cnt + 1
            return carry2

        lax.fori_loop(0, nhv, hit_body, 0)

    def issue(t, slot):
        @pl.when(t < full)
        def _():
            base = pl.multiple_of(t * _SSW, _SSW)
            pltpu.async_copy(tbl_h.at[:, pl.ds(base, _SSW)],
                             ring_v.at[slot], sem)

    for k in range(3):
        issue(wid + k * _NW, k)

    nt = (full - wid + _NW - 1) // _NW

    def slab_body(k, carry):
        t = wid + k * _NW
        slot = k % 3
        pltpu.make_async_copy(tbl_h.at[:, pl.ds(0, _SSW)],
                              ring_v.at[slot], sem).wait()
        match_hits(t, slot)
        issue(wid + (k + 3) * _NW, slot)
        return carry

    lax.fori_loop(0, nt, slab_body, 0)

    if tw:
        twp = tail_h.shape[1]

        @pl.when(wid == full % _NW)
        def _():
            pltpu.sync_copy(tail_h, ring_v.at[0, :, pl.ds(0, twp)])
            match_hits(full, 0)


def _ka_body(wt_h, ht_h, h1t_h, wtail_h, htail_h, h1tail_h,
             u_h, i_h, j_h, i1_h,
             ue_h, ieje_h, a_h,
             hb_v, hr_v, ring_v, idxc_v, stage_v, drain_v, cnt_s,
             sem, sem2):
    wid = lax.axis_index("s") * _NC + lax.axis_index("c")
    cnt_s[0] = 0
    dumprows = jnp.full((16,), _B + wid, jnp.int32)
    for k in range(4):
        pltpu.async_copy(stage_v.at[k], ue_h.at[dumprows], sem2)
    common = (wid, hb_v, hr_v, ring_v, idxc_v, stage_v, drain_v, cnt_s,
              sem, sem2)
    _scan_pass(wt_h, wtail_h, 1000000, [u_h], ue_h, *common)
    _scan_pass(ht_h, htail_h, 1000000, [i_h, j_h], ieje_h, *common)
    _scan_pass(h1t_h, h1tail_h, 100000, [i1_h], a_h, *common)
    for _ in range(4):
        pltpu.make_async_copy(ue_h.at[pl.ds(_B, 16)], drain_v, sem2).wait()


def _kb_body(ue_h, ieje_h, a_h, h2t_h, i2_h, r1_h, r2_h, out_h,
             ue_v, ie_v, je_v, a_v, h2_v, i2_v, r1_v, r2_v, p_v, sem):
    wid = lax.axis_index("s") * _NC + lax.axis_index("c")
    pltpu.sync_copy(h2t_h, h2_v)
    for k in range(4):
        pltpu.sync_copy(i2_h.at[wid * 4 + k], i2_v.at[pl.ds(k * 128, 128)])
        pltpu.sync_copy(r1_h.at[wid * 4 + k], r1_v.at[pl.ds(k * 128, 128)])
        pltpu.sync_copy(r2_h.at[wid * 4 + k], r2_v.at[pl.ds(k * 128, 128)])

    iota16 = lax.iota(jnp.int32, 16)
    zero = jnp.zeros((16,), jnp.float32)
    acc_total = zero

    for s in range(4):
        base = pl.multiple_of(wid * _BPW + s * 128, 128)
        copies = [
            pltpu.async_copy(ue_h.at[pl.ds(base, 128)], ue_v, sem),
            pltpu.async_copy(ieje_h.at[pl.ds(base, 128)], ie_v, sem),
            pltpu.async_copy(ieje_h.at[pl.ds(_ROWS_PAD + base, 128)],
                             je_v, sem),
            pltpu.async_copy(a_h.at[pl.ds(base, 128)], a_v, sem),
        ]
        for cp in copies:
            cp.wait()

        def block(b, acc, s=s):
            roff = pl.multiple_of(b * 16, 16)
            rvec = iota16 + roff
            goff = pl.multiple_of(s * 128 + b * 16, 16)
            i2vals = i2_v[pl.ds(goff, 16)]
            x = x1 = x2 = ru = q1 = q2 = zero
            for c in range(_D):
                cv = jnp.full((16,), c, jnp.int32)
                ue = plsc.load_gather(ue_v, [rvec, cv])
                ie = plsc.load_gather(ie_v, [rvec, cv])
                je = plsc.load_gather(je_v, [rvec, cv])
                ae = plsc.load_gather(a_v, [rvec, cv])
                ge = plsc.load_gather(h2_v, [cv, i2vals])
                x = x + ue * (ie - je)
                x1 = x1 + ue * ae
                x2 = x2 + ue * ge
                ru = ru + ue * ue
                d1 = ae - ie
                q1 = q1 + d1 * d1
                d2 = ge - ae
                q2 = q2 + d2 * d2
            ls = _log_sigmoid(x)
            t1 = r1_v[pl.ds(goff, 16)] - x1
            t2 = r2_v[pl.ds(goff, 16)] - x2
            return acc + (-ls + t1 * t1 + t2 * t2
                          + jnp.float32(_LAMBDA) * (ru + q1 + q2))

        acc_total = lax.fori_loop(0, 8, block, acc_total)

    p_v[...] = acc_total
    pltpu.sync_copy(p_v, out_h.at[wid])


def _tc_finish(p_ref, o_ref):
    o_ref[0, 0] = jnp.sum(p_ref[...])


@jax.jit
def _run(u2, i2, j2, i1_2, i2_2, W, H, H_1, H_2, r1_2, r2_2):
    mesh = plsc.VectorSubcoreMesh(core_axis_name="c", subcore_axis_name="s")
    params = pltpu.CompilerParams(needs_layout_passes=False)
    ka = pl.kernel(
        _ka_body,
        out_type=(
            jax.ShapeDtypeStruct((_ROWS_PAD, 128), jnp.float32),
            jax.ShapeDtypeStruct((2 * _ROWS_PAD, 128), jnp.float32),
            jax.ShapeDtypeStruct((_ROWS_PAD, 128), jnp.float32),
        ),
        mesh=mesh,
        compiler_params=params,
        scratch_types=[
            pltpu.VMEM((2 * _B + 32,), jnp.int32),
            pltpu.VMEM((2 * _B + 32,), jnp.int32),
            pltpu.VMEM((3, 32, _SSW), jnp.float32),
            pltpu.VMEM((2, 128), jnp.int32),
            pltpu.VMEM((5, 16, 128), jnp.float32),
            pltpu.VMEM((16, 128), jnp.float32),
            pltpu.SMEM((1,), jnp.int32),
            pltpu.SemaphoreType.DMA,
            pltpu.SemaphoreType.DMA,
        ],
    )
    wtail = jnp.pad(W.T[:, 999936:], ((0, 0), (0, 64)))
    htail = jnp.pad(H.T[:, 999936:], ((0, 0), (0, 64)))
    h1tail = jnp.pad(H_1.T[:, 99840:], ((0, 0), (0, 96)))
    ue_buf, ieje_buf, a_buf = ka(W.T, H.T, H_1.T, wtail, htail, h1tail,
                                 u2, i2, j2, i1_2)

    kb = pl.kernel(
        _kb_body,
        out_type=jax.ShapeDtypeStruct((_NW, 16), jnp.float32),
        mesh=mesh,
        compiler_params=params,
        scratch_types=[
            pltpu.VMEM((128, 128), jnp.float32),
            pltpu.VMEM((128, 128), jnp.float32),
            pltpu.VMEM((128, 128), jnp.float32),
            pltpu.VMEM((128, 128), jnp.float32),
            pltpu.VMEM((32, 1000), jnp.float32),
            pltpu.VMEM((_BPW,), jnp.int32),
            pltpu.VMEM((_BPW,), jnp.float32),
            pltpu.VMEM((_BPW,), jnp.float32),
            pltpu.VMEM((16,), jnp.float32),
            pltpu.SemaphoreType.DMA,
        ],
    )
    partials = kb(ue_buf, ieje_buf, a_buf, H_2.T, i2_2, r1_2, r2_2)

    total = pl.pallas_call(
        _tc_finish,
        out_shape=jax.ShapeDtypeStruct((1, 1), jnp.float32),
        out_specs=pl.BlockSpec(memory_space=pltpu.SMEM),
    )(partials)
    return total[0, 0]


def kernel(u, i, j, i_1, i_2, batch_size, W, H, H_1, H_2, r_1, r_2):
    u2 = u.astype(jnp.int32).reshape(128, 128)
    i2 = i.astype(jnp.int32).reshape(128, 128)
    j2 = j.astype(jnp.int32).reshape(128, 128)
    i1_2 = i_1.astype(jnp.int32).reshape(128, 128)
    i2_2 = i_2.astype(jnp.int32).reshape(128, 128)
    r1_2 = r_1.reshape(128, 128)
    r2_2 = r_2.reshape(128, 128)
    total = _run(u2, i2, j2, i1_2, i2_2, W, H, H_1, H_2, r1_2, r2_2)
    return total / batch_size


# speedup vs baseline: 2.2594x; 2.1835x over previous
import jax
import jax.numpy as jnp
from jax import lax
from jax.experimental import pallas as pl
from jax.experimental.pallas import tpu as pltpu
from jax.experimental.pallas import tpu_sc as plsc

_B = 16384
_D = 32
_NC = 2
_NS = 16
_NW = _NC * _NS
_BPW = _B // _NW
_SSW = 512
_ROWS_PAD = _B + _NW
_LAMBDA = 0.01

_LOG_COEFFS = (
    7.0376836292e-2, -1.1514610310e-1, 1.1676998740e-1, -1.2420140846e-1,
    1.4249322787e-1, -1.6668057665e-1, 2.0000714765e-1, -2.4999993993e-1,
    3.3333331174e-1,
)


def _logf(y):
    bits = lax.bitcast_convert_type(y, jnp.int32)
    e = lax.shift_right_logical(bits, 23) - 126
    m = lax.bitcast_convert_type(
        jnp.bitwise_or(jnp.bitwise_and(bits, 0x007FFFFF), 0x3F000000),
        jnp.float32)
    big = m > jnp.float32(0.70710678)
    e = jnp.where(big, e, e - 1).astype(jnp.float32)
    x = jnp.where(big, m - 1.0, m + m - 1.0)
    z = x * x
    p = jnp.full_like(x, _LOG_COEFFS[0])
    for c in _LOG_COEFFS[1:]:
        p = p * x + c
    r = x * z * p
    r = r + e * jnp.float32(-2.12194440e-4)
    r = r - jnp.float32(0.5) * z
    return x + r + e * jnp.float32(0.693359375)


def _log_sigmoid(x):
    t = jnp.exp(-jnp.abs(x))
    return jnp.minimum(x, 0.0) - _logf(1.0 + t)


def _scan_pass(tbl_h, tail_h, n_rows, idx_sets, out_h, wid,
               hb_v, hr_v, ring_v, idxc_v, stage_v, drain_v,
               hist_v, base_v, off_v, cnt_s, sem, sem2):
    full = n_rows // _SSW
    tw = n_rows - full * _SSW
    iota16 = lax.iota(jnp.int32, 16)
    dump = jnp.full((16,), _B + wid, jnp.int32)

    zeros16 = jnp.zeros((16,), jnp.int32)
    for wl in range(16):
        for g in range(4):
            hist_v[wl, pl.ds(g * 16, 16)] = zeros16

    def sweep(set_id, idx_h, body):
        pltpu.async_copy(idx_h.at[0], idxc_v.at[0], sem)

        def blk_body(blk, carry):
            slot = blk % 2
            pltpu.make_async_copy(idx_h.at[0], idxc_v.at[slot], sem).wait()

            @pl.when(blk < 127)
            def _():
                pltpu.async_copy(idx_h.at[blk + 1],
                                 idxc_v.at[1 - slot], sem)
            for q in range(8):
                v = idxc_v[slot, pl.ds(q * 16, 16)]
                m = jnp.bitwise_and(
                    lax.shift_right_logical(v, 9), _NW - 1) == wid
                body(blk, q, set_id, v, m)
            return carry
        lax.fori_loop(0, 128, blk_body, 0)

    def count_body(blk, q, set_id, v, m):
        l = lax.shift_right_logical(v, 14)
        plsc.addupdate_scatter(hist_v, [iota16, l], m.astype(jnp.int32))

    for set_id, idx_h in enumerate(idx_sets):
        sweep(set_id, idx_h, count_body)

    carry = jnp.int32(0)
    for g in range(4):
        colsum = zeros16
        for wl in range(16):
            colsum = colsum + hist_v[wl, pl.ds(g * 16, 16)]
        padded = jnp.bitwise_and(colsum + 15, ~15)
        inc = plsc.cumsum(padded)
        ex = inc - padded + carry
        off_v[pl.ds(g * 16, 16)] = ex
        run = ex
        for wl in range(16):
            base_v[wl, pl.ds(g * 16, 16)] = run
            run = run + hist_v[wl, pl.ds(g * 16, 16)]
        carry = carry + jnp.max(inc)

    sent = jnp.full((16,), -1, jnp.int32)

    def fill_body(z, c):
        hr_v[pl.ds(pl.multiple_of(z * 16, 16), 16)] = sent
        return c
    lax.fori_loop(0, lax.shift_right_logical(carry + 15, 4), fill_body, 0)

    def place_body(blk, q, set_id, v, m):
        l = lax.shift_right_logical(v, 14)
        pos = plsc.load_gather(base_v, [iota16, l])
        b_enc = blk * 128 + (q * 16 + set_id * _B) + iota16
        plsc.store_scatter(hb_v, [pos], b_enc, mask=m)
        plsc.store_scatter(hr_v, [pos], v, mask=m)
        plsc.addupdate_scatter(base_v, [iota16, l], m.astype(jnp.int32))

    for set_id, idx_h in enumerate(idx_sets):
        sweep(set_id, idx_h, place_body)

    def extract(h, slot, t):
        off = pl.multiple_of(h * 16, 16)
        hbv = hb_v[pl.ds(off, 16)]
        hrv = hr_v[pl.ds(off, 16)]
        m = lax.shift_right_logical(hrv, 9) == t
        cnt = cnt_s[0]
        sslot = lax.rem(cnt, 4)
        pltpu.make_async_copy(out_h.at[pl.ds(_B, 16)], drain_v, sem2).wait()
        p = jnp.bitwise_and(hrv, _SSW - 1)
        sv = jnp.full((16,), sslot, jnp.int32)
        for c in range(_D):
            val = plsc.load_gather(
                ring_v.at[slot], [jnp.full((16,), c, jnp.int32), p])
            plsc.store_scatter(
                stage_v, [sv, iota16, jnp.full((16,), c, jnp.int32)],
                val, mask=m)
        rows = (jnp.bitwise_and(hbv, _B - 1)
                + lax.shift_right_logical(hbv, 14) * _ROWS_PAD)
        rows = jnp.where(m, rows, dump)
        pltpu.async_copy(stage_v.at[sslot], out_h.at[rows], sem2)
        cnt_s[0] = cnt + 1

    def bucket_loop(k, slot, t):
        off_s = jnp.max(plsc.load_gather(off_v, [jnp.full((16,), k,
                                                          jnp.int32)]))
        off_e = jnp.max(plsc.load_gather(off_v, [jnp.full((16,), k + 1,
                                                          jnp.int32)]))

        def hit_body(h, c2):
            extract(h, slot, t)
            return c2
        lax.fori_loop(lax.shift_right_logical(off_s, 4),
                      lax.shift_right_logical(off_e, 4), hit_body, 0)

    def issue(t, slot):
        @pl.when(t < full)
        def _():
            base = pl.multiple_of(t * _SSW, _SSW)
            pltpu.async_copy(tbl_h.at[:, pl.ds(base, _SSW)],
                             ring_v.at[slot], sem)

    for k in range(3):
        issue(wid + k * _NW, k)

    nt = (full - wid + _NW - 1) // _NW

    def slab_body(k, carry2):
        t = wid + k * _NW
        slot = k % 3
        pltpu.make_async_copy(tbl_h.at[:, pl.ds(0, _SSW)],
                              ring_v.at[slot], sem).wait()
        bucket_loop(k, slot, t)
        issue(wid + (k + 3) * _NW, slot)
        return carry2

    lax.fori_loop(0, nt, slab_body, 0)

    if tw:
        twp = tail_h.shape[1]

        @pl.when(wid == full % _NW)
        def _():
            pltpu.sync_copy(tail_h, ring_v.at[0, :, pl.ds(0, twp)])
            bucket_loop(full // _NW, 0, full)


def _ka_body(wt_h, ht_h, h1t_h, wtail_h, htail_h, h1tail_h,
             u_h, i_h, j_h, i1_h,
             ue_h, ieje_h, a_h,
             hb_v, hr_v, ring_v, idxc_v, stage_v, drain_v,
             hist_v, base_v, off_v, cnt_s, sem, sem2):
    wid = lax.axis_index("s") * _NC + lax.axis_index("c")
    cnt_s[0] = 0
    dumprows = jnp.full((16,), _B + wid, jnp.int32)
    for k in range(3):
        pltpu.async_copy(stage_v.at[k], ue_h.at[dumprows], sem2)
    common = (wid, hb_v, hr_v, ring_v, idxc_v, stage_v, drain_v,
              hist_v, base_v, off_v, cnt_s, sem, sem2)
    _scan_pass(wt_h, wtail_h, 1000000, [u_h], ue_h, *common)
    _scan_pass(ht_h, htail_h, 1000000, [i_h, j_h], ieje_h, *common)
    _scan_pass(h1t_h, h1tail_h, 100000, [i1_h], a_h, *common)
    for _ in range(3):
        pltpu.make_async_copy(ue_h.at[pl.ds(_B, 16)], drain_v, sem2).wait()


def _kb_body(ue_h, ieje_h, a_h, h2t_h, i2_h, r1_h, r2_h, out_h,
             ue_v, ie_v, je_v, a_v, h2_v, i2_v, r1_v, r2_v, p_v, sem):
    wid = lax.axis_index("s") * _NC + lax.axis_index("c")
    pltpu.sync_copy(h2t_h, h2_v)
    for k in range(4):
        pltpu.sync_copy(i2_h.at[wid * 4 + k], i2_v.at[pl.ds(k * 128, 128)])
        pltpu.sync_copy(r1_h.at[wid * 4 + k], r1_v.at[pl.ds(k * 128, 128)])
        pltpu.sync_copy(r2_h.at[wid * 4 + k], r2_v.at[pl.ds(k * 128, 128)])

    iota16 = lax.iota(jnp.int32, 16)
    zero = jnp.zeros((16,), jnp.float32)
    acc_total = zero

    for s in range(4):
        base = pl.multiple_of(wid * _BPW + s * 128, 128)
        copies = [
            pltpu.async_copy(ue_h.at[pl.ds(base, 128)], ue_v, sem),
            pltpu.async_copy(ieje_h.at[pl.ds(base, 128)], ie_v, sem),
            pltpu.async_copy(ieje_h.at[pl.ds(_ROWS_PAD + base, 128)],
                             je_v, sem),
            pltpu.async_copy(a_h.at[pl.ds(base, 128)], a_v, sem),
        ]
        for cp in copies:
            cp.wait()

        def block(b, acc, s=s):
            roff = pl.multiple_of(b * 16, 16)
            rvec = iota16 + roff
            goff = pl.multiple_of(s * 128 + b * 16, 16)
            i2vals = i2_v[pl.ds(goff, 16)]
            x = x1 = x2 = ru = q1 = q2 = zero
            for c in range(_D):
                cv = jnp.full((16,), c, jnp.int32)
                ue = plsc.load_gather(ue_v, [rvec, cv])
                ie = plsc.load_gather(ie_v, [rvec, cv])
                je = plsc.load_gather(je_v, [rvec, cv])
                ae = plsc.load_gather(a_v, [rvec, cv])
                ge = plsc.load_gather(h2_v, [cv, i2vals])
                x = x + ue * (ie - je)
                x1 = x1 + ue * ae
                x2 = x2 + ue * ge
                ru = ru + ue * ue
                d1 = ae - ie
                q1 = q1 + d1 * d1
                d2 = ge - ae
                q2 = q2 + d2 * d2
            ls = _log_sigmoid(x)
            t1 = r1_v[pl.ds(goff, 16)] - x1
            t2 = r2_v[pl.ds(goff, 16)] - x2
            return acc + (-ls + t1 * t1 + t2 * t2
                          + jnp.float32(_LAMBDA) * (ru + q1 + q2))

        acc_total = lax.fori_loop(0, 8, block, acc_total)

    p_v[...] = acc_total
    pltpu.sync_copy(p_v, out_h.at[wid])


def _tc_finish(p_ref, o_ref):
    o_ref[0, 0] = jnp.sum(p_ref[...])


@jax.jit
def _run(u2, i2, j2, i1_2, i2_2, W, H, H_1, H_2, r1_2, r2_2):
    mesh = plsc.VectorSubcoreMesh(core_axis_name="c", subcore_axis_name="s")
    params = pltpu.CompilerParams(needs_layout_passes=False)
    ka = pl.kernel(
        _ka_body,
        out_type=(
            jax.ShapeDtypeStruct((_ROWS_PAD, 128), jnp.float32),
            jax.ShapeDtypeStruct((2 * _ROWS_PAD, 128), jnp.float32),
            jax.ShapeDtypeStruct((_ROWS_PAD, 128), jnp.float32),
        ),
        mesh=mesh,
        compiler_params=params,
        scratch_types=[
            pltpu.VMEM((2 * _B + 960,), jnp.int32),
            pltpu.VMEM((2 * _B + 960,), jnp.int32),
            pltpu.VMEM((3, 32, _SSW), jnp.float32),
            pltpu.VMEM((2, 128), jnp.int32),
            pltpu.VMEM((4, 16, 128), jnp.float32),
            pltpu.VMEM((16, 128), jnp.float32),
            pltpu.VMEM((16, 64), jnp.int32),
            pltpu.VMEM((16, 64), jnp.int32),
            pltpu.VMEM((80,), jnp.int32),
            pltpu.SMEM((1,), jnp.int32),
            pltpu.SemaphoreType.DMA,
            pltpu.SemaphoreType.DMA,
        ],
    )
    wtail = jnp.pad(W.T[:, 999936:], ((0, 0), (0, 64)))
    htail = jnp.pad(H.T[:, 999936:], ((0, 0), (0, 64)))
    h1tail = jnp.pad(H_1.T[:, 99840:], ((0, 0), (0, 96)))
    ue_buf, ieje_buf, a_buf = ka(W.T, H.T, H_1.T, wtail, htail, h1tail,
                                 u2, i2, j2, i1_2)

    kb = pl.kernel(
        _kb_body,
        out_type=jax.ShapeDtypeStruct((_NW, 16), jnp.float32),
        mesh=mesh,
        compiler_params=params,
        scratch_types=[
            pltpu.VMEM((128, 128), jnp.float32),
            pltpu.VMEM((128, 128), jnp.float32),
            pltpu.VMEM((128, 128), jnp.float32),
            pltpu.VMEM((128, 128), jnp.float32),
            pltpu.VMEM((32, 1000), jnp.float32),
            pltpu.VMEM((_BPW,), jnp.int32),
            pltpu.VMEM((_BPW,), jnp.float32),
            pltpu.VMEM((_BPW,), jnp.float32),
            pltpu.VMEM((16,), jnp.float32),
            pltpu.SemaphoreType.DMA,
        ],
    )
    partials = kb(ue_buf, ieje_buf, a_buf, H_2.T, i2_2, r1_2, r2_2)

    total = pl.pallas_call(
        _tc_finish,
        out_shape=jax.ShapeDtypeStruct((1, 1), jnp.float32),
        out_specs=pl.BlockSpec(memory_space=pltpu.SMEM),
    )(partials)
    return total[0, 0]


def kernel(u, i, j, i_1, i_2, batch_size, W, H, H_1, H_2, r_1, r_2):
    u2 = u.astype(jnp.int32).reshape(128, 128)
    i2 = i.astype(jnp.int32).reshape(128, 128)
    j2 = j.astype(jnp.int32).reshape(128, 128)
    i1_2 = i_1.astype(jnp.int32).reshape(128, 128)
    i2_2 = i_2.astype(jnp.int32).reshape(128, 128)
    r1_2 = r_1.reshape(128, 128)
    r2_2 = r_2.reshape(128, 128)
    total = _run(u2, i2, j2, i1_2, i2_2, W, H, H_1, H_2, r1_2, r2_2)
    return total / batch_size


# speedup vs baseline: 4.3250x; 1.9142x over previous
import jax
import jax.numpy as jnp
from jax import lax
from jax.experimental import pallas as pl
from jax.experimental.pallas import tpu as pltpu
from jax.experimental.pallas import tpu_sc as plsc

_B = 16384
_D = 32
_NC = 2
_NS = 16
_NW = _NC * _NS
_BPW = _B // _NW
_SSW = 512
_ROWS_PAD = _B + _NW
_LAMBDA = 0.01

_LOG_COEFFS = (
    7.0376836292e-2, -1.1514610310e-1, 1.1676998740e-1, -1.2420140846e-1,
    1.4249322787e-1, -1.6668057665e-1, 2.0000714765e-1, -2.4999993993e-1,
    3.3333331174e-1,
)


def _logf(y):
    bits = lax.bitcast_convert_type(y, jnp.int32)
    e = lax.shift_right_logical(bits, 23) - 126
    m = lax.bitcast_convert_type(
        jnp.bitwise_or(jnp.bitwise_and(bits, 0x007FFFFF), 0x3F000000),
        jnp.float32)
    big = m > jnp.float32(0.70710678)
    e = jnp.where(big, e, e - 1).astype(jnp.float32)
    x = jnp.where(big, m - 1.0, m + m - 1.0)
    z = x * x
    p = jnp.full_like(x, _LOG_COEFFS[0])
    for c in _LOG_COEFFS[1:]:
        p = p * x + c
    r = x * z * p
    r = r + e * jnp.float32(-2.12194440e-4)
    r = r - jnp.float32(0.5) * z
    return x + r + e * jnp.float32(0.693359375)


def _log_sigmoid(x):
    t = jnp.exp(-jnp.abs(x))
    return jnp.minimum(x, 0.0) - _logf(1.0 + t)


def _scan_pass(tbl_h, tail_h, n_rows, idx_sets, out_h, wid,
               hb_v, hr_v, ring_v, stage_v, drain_v,
               hist_v, base_v, off_v, cnt_s, sem, sem2):
    full = n_rows // _SSW
    tw = n_rows - full * _SSW
    iota16 = lax.iota(jnp.int32, 16)
    dump = jnp.full((16,), _B + wid, jnp.int32)

    zeros16 = jnp.zeros((16,), jnp.int32)
    for wl in range(16):
        for g in range(4):
            hist_v[wl, pl.ds(g * 16, 16)] = zeros16

    def load_idx(idx_h, slot):
        for k in range(4):
            pltpu.sync_copy(idx_h.at[pl.ds(32 * k, 32)],
                            ring_v.at[slot, :, pl.ds(128 * k, 128)])

    for set_id, idx_h in enumerate(idx_sets):
        load_idx(idx_h, set_id)

    def sweep(set_id, body):
        def blk_body(blk, carry):
            rr = lax.rem(blk, 32)
            cc = pl.multiple_of(lax.div(blk, 32) * 128, 128)
            for q in range(8):
                vf = ring_v[set_id, rr, pl.ds(cc + q * 16, 16)]
                v = plsc.bitcast(vf, jnp.int32)
                m = jnp.bitwise_and(
                    lax.shift_right_logical(v, 9), _NW - 1) == wid
                body(blk, q, set_id, v, m)
            return carry
        lax.fori_loop(0, 128, blk_body, 0)

    def count_body(blk, q, set_id, v, m):
        l = lax.shift_right_logical(v, 14)
        plsc.addupdate_scatter(hist_v, [iota16, l], m.astype(jnp.int32))

    for set_id in range(len(idx_sets)):
        sweep(set_id, count_body)

    carry = jnp.int32(0)
    for g in range(4):
        colsum = zeros16
        for wl in range(16):
            colsum = colsum + hist_v[wl, pl.ds(g * 16, 16)]
        padded = jnp.bitwise_and(colsum + 15, ~15)
        inc = plsc.cumsum(padded)
        ex = inc - padded + carry
        off_v[pl.ds(g * 16, 16)] = ex
        run = ex
        for wl in range(16):
            base_v[wl, pl.ds(g * 16, 16)] = run
            run = run + hist_v[wl, pl.ds(g * 16, 16)]
        carry = carry + jnp.max(inc)

    sent = jnp.full((16,), -1, jnp.int32)

    def fill_body(z, c):
        hr_v[pl.ds(pl.multiple_of(z * 16, 16), 16)] = sent
        return c
    lax.fori_loop(0, lax.shift_right_logical(carry + 15, 4), fill_body, 0)

    def place_body(blk, q, set_id, v, m):
        l = lax.shift_right_logical(v, 14)
        pos = plsc.load_gather(base_v, [iota16, l])
        b_enc = blk * 128 + (q * 16 + set_id * _B) + iota16
        plsc.store_scatter(hb_v, [pos], b_enc, mask=m)
        plsc.store_scatter(hr_v, [pos], v, mask=m)
        plsc.addupdate_scatter(base_v, [iota16, l], m.astype(jnp.int32))

    for set_id in range(len(idx_sets)):
        sweep(set_id, place_body)

    def extract(h, slot, t):
        off = pl.multiple_of(h * 16, 16)
        hbv = hb_v[pl.ds(off, 16)]
        hrv = hr_v[pl.ds(off, 16)]
        m = lax.shift_right_logical(hrv, 9) == t
        cnt = cnt_s[0]
        sslot = lax.rem(cnt, 4)
        pltpu.make_async_copy(out_h.at[pl.ds(_B, 16)], drain_v, sem2).wait()
        p = jnp.bitwise_and(hrv, _SSW - 1)
        sv = jnp.full((16,), sslot, jnp.int32)
        for c in range(_D):
            val = plsc.load_gather(
                ring_v.at[slot], [jnp.full((16,), c, jnp.int32), p])
            plsc.store_scatter(
                stage_v, [sv, iota16, jnp.full((16,), c, jnp.int32)],
                val, mask=m)
        rows = (jnp.bitwise_and(hbv, _B - 1)
                + lax.shift_right_logical(hbv, 14) * _ROWS_PAD)
        rows = jnp.where(m, rows, dump)
        pltpu.async_copy(stage_v.at[sslot], out_h.at[rows], sem2)
        cnt_s[0] = cnt + 1

    def bucket_loop(k, slot, t):
        off_s = jnp.max(plsc.load_gather(off_v, [jnp.full((16,), k,
                                                          jnp.int32)]))
        off_e = jnp.max(plsc.load_gather(off_v, [jnp.full((16,), k + 1,
                                                          jnp.int32)]))

        def hit_body(h, c2):
            extract(h, slot, t)
            return c2
        lax.fori_loop(lax.shift_right_logical(off_s, 4),
                      lax.shift_right_logical(off_e, 4), hit_body, 0)

    def issue(t, slot):
        @pl.when(t < full)
        def _():
            base = pl.multiple_of(t * _SSW, _SSW)
            pltpu.async_copy(tbl_h.at[:, pl.ds(base, _SSW)],
                             ring_v.at[slot], sem)

    for k in range(3):
        issue(wid + k * _NW, k)

    nt = (full - wid + _NW - 1) // _NW

    def slab_body(k, carry2):
        t = wid + k * _NW
        slot = k % 3
        pltpu.make_async_copy(tbl_h.at[:, pl.ds(0, _SSW)],
                              ring_v.at[slot], sem).wait()
        bucket_loop(k, slot, t)
        issue(wid + (k + 3) * _NW, slot)
        return carry2

    lax.fori_loop(0, nt, slab_body, 0)

    if tw:
        twp = tail_h.shape[1]

        @pl.when(wid == full % _NW)
        def _():
            pltpu.sync_copy(tail_h, ring_v.at[0, :, pl.ds(0, twp)])
            bucket_loop(full // _NW, 0, full)


def _ka_body(wt_h, ht_h, h1t_h, wtail_h, htail_h, h1tail_h,
             u_h, i_h, j_h, i1_h,
             ue_h, ieje_h, a_h,
             hb_v, hr_v, ring_v, stage_v, drain_v,
             hist_v, base_v, off_v, cnt_s, sem, sem2):
    wid = lax.axis_index("s") * _NC + lax.axis_index("c")
    cnt_s[0] = 0
    dumprows = jnp.full((16,), _B + wid, jnp.int32)
    for k in range(3):
        pltpu.async_copy(stage_v.at[k], ue_h.at[dumprows], sem2)
    common = (wid, hb_v, hr_v, ring_v, stage_v, drain_v,
              hist_v, base_v, off_v, cnt_s, sem, sem2)
    _scan_pass(wt_h, wtail_h, 1000000, [u_h], ue_h, *common)
    _scan_pass(ht_h, htail_h, 1000000, [i_h, j_h], ieje_h, *common)
    _scan_pass(h1t_h, h1tail_h, 100000, [i1_h], a_h, *common)
    for _ in range(3):
        pltpu.make_async_copy(ue_h.at[pl.ds(_B, 16)], drain_v, sem2).wait()


def _kb_body(ue_h, ieje_h, a_h, h2t_h, i2_h, r1_h, r2_h, out_h,
             ue_v, ie_v, je_v, a_v, h2_v, i2_v, r1_v, r2_v, p_v, sem):
    wid = lax.axis_index("s") * _NC + lax.axis_index("c")
    pltpu.sync_copy(h2t_h, h2_v)
    for k in range(4):
        pltpu.sync_copy(i2_h.at[wid * 4 + k], i2_v.at[pl.ds(k * 128, 128)])
        pltpu.sync_copy(r1_h.at[wid * 4 + k], r1_v.at[pl.ds(k * 128, 128)])
        pltpu.sync_copy(r2_h.at[wid * 4 + k], r2_v.at[pl.ds(k * 128, 128)])

    iota16 = lax.iota(jnp.int32, 16)
    zero = jnp.zeros((16,), jnp.float32)
    acc_total = zero

    for s in range(4):
        base = pl.multiple_of(wid * _BPW + s * 128, 128)
        copies = [
            pltpu.async_copy(ue_h.at[pl.ds(base, 128)], ue_v, sem),
            pltpu.async_copy(ieje_h.at[pl.ds(base, 128)], ie_v, sem),
            pltpu.async_copy(ieje_h.at[pl.ds(_ROWS_PAD + base, 128)],
                             je_v, sem),
            pltpu.async_copy(a_h.at[pl.ds(base, 128)], a_v, sem),
        ]
        for cp in copies:
            cp.wait()

        def block(b, acc, s=s):
            roff = pl.multiple_of(b * 16, 16)
            rvec = iota16 + roff
            goff = pl.multiple_of(s * 128 + b * 16, 16)
            i2vals = i2_v[pl.ds(goff, 16)]
            x = x1 = x2 = ru = q1 = q2 = zero
            for c in range(_D):
                cv = jnp.full((16,), c, jnp.int32)
                ue = plsc.load_gather(ue_v, [rvec, cv])
                ie = plsc.load_gather(ie_v, [rvec, cv])
                je = plsc.load_gather(je_v, [rvec, cv])
                ae = plsc.load_gather(a_v, [rvec, cv])
                ge = plsc.load_gather(h2_v, [cv, i2vals])
                x = x + ue * (ie - je)
                x1 = x1 + ue * ae
                x2 = x2 + ue * ge
                ru = ru + ue * ue
                d1 = ae - ie
                q1 = q1 + d1 * d1
                d2 = ge - ae
                q2 = q2 + d2 * d2
            ls = _log_sigmoid(x)
            t1 = r1_v[pl.ds(goff, 16)] - x1
            t2 = r2_v[pl.ds(goff, 16)] - x2
            return acc + (-ls + t1 * t1 + t2 * t2
                          + jnp.float32(_LAMBDA) * (ru + q1 + q2))

        acc_total = lax.fori_loop(0, 8, block, acc_total)

    p_v[...] = acc_total
    pltpu.sync_copy(p_v, out_h.at[wid])


def _tc_finish(p_ref, o_ref):
    o_ref[0, 0] = jnp.sum(p_ref[...])


@jax.jit
def _run(u2, i2, j2, i1_2, i2_2, W, H, H_1, H_2, r1_2, r2_2):
    mesh = plsc.VectorSubcoreMesh(core_axis_name="c", subcore_axis_name="s")
    params = pltpu.CompilerParams(needs_layout_passes=False)
    ka = pl.kernel(
        _ka_body,
        out_type=(
            jax.ShapeDtypeStruct((_ROWS_PAD, 128), jnp.float32),
            jax.ShapeDtypeStruct((2 * _ROWS_PAD, 128), jnp.float32),
            jax.ShapeDtypeStruct((_ROWS_PAD, 128), jnp.float32),
        ),
        mesh=mesh,
        compiler_params=params,
        scratch_types=[
            pltpu.VMEM((2 * _B + 960,), jnp.int32),
            pltpu.VMEM((2 * _B + 960,), jnp.int32),
            pltpu.VMEM((3, 32, _SSW), jnp.float32),
            pltpu.VMEM((4, 16, 128), jnp.float32),
            pltpu.VMEM((16, 128), jnp.float32),
            pltpu.VMEM((16, 64), jnp.int32),
            pltpu.VMEM((16, 64), jnp.int32),
            pltpu.VMEM((80,), jnp.int32),
            pltpu.SMEM((1,), jnp.int32),
            pltpu.SemaphoreType.DMA,
            pltpu.SemaphoreType.DMA,
        ],
    )
    wtail = jnp.pad(W.T[:, 999936:], ((0, 0), (0, 64)))
    htail = jnp.pad(H.T[:, 999936:], ((0, 0), (0, 64)))
    h1tail = jnp.pad(H_1.T[:, 99840:], ((0, 0), (0, 96)))
    uf = lax.bitcast_convert_type(u2, jnp.float32)
    if_ = lax.bitcast_convert_type(i2, jnp.float32)
    jf = lax.bitcast_convert_type(j2, jnp.float32)
    i1f = lax.bitcast_convert_type(i1_2, jnp.float32)
    ue_buf, ieje_buf, a_buf = ka(W.T, H.T, H_1.T, wtail, htail, h1tail,
                                 uf, if_, jf, i1f)

    kb = pl.kernel(
        _kb_body,
        out_type=jax.ShapeDtypeStruct((_NW, 16), jnp.float32),
        mesh=mesh,
        compiler_params=params,
        scratch_types=[
            pltpu.VMEM((128, 128), jnp.float32),
            pltpu.VMEM((128, 128), jnp.float32),
            pltpu.VMEM((128, 128), jnp.float32),
            pltpu.VMEM((128, 128), jnp.float32),
            pltpu.VMEM((32, 1000), jnp.float32),
            pltpu.VMEM((_BPW,), jnp.int32),
            pltpu.VMEM((_BPW,), jnp.float32),
            pltpu.VMEM((_BPW,), jnp.float32),
            pltpu.VMEM((16,), jnp.float32),
            pltpu.SemaphoreType.DMA,
        ],
    )
    partials = kb(ue_buf, ieje_buf, a_buf, H_2.T, i2_2, r1_2, r2_2)

    total = pl.pallas_call(
        _tc_finish,
        out_shape=jax.ShapeDtypeStruct((1, 1), jnp.float32),
        out_specs=pl.BlockSpec(memory_space=pltpu.SMEM),
    )(partials)
    return total[0, 0]


def kernel(u, i, j, i_1, i_2, batch_size, W, H, H_1, H_2, r_1, r_2):
    u2 = u.astype(jnp.int32).reshape(128, 128)
    i2 = i.astype(jnp.int32).reshape(128, 128)
    j2 = j.astype(jnp.int32).reshape(128, 128)
    i1_2 = i_1.astype(jnp.int32).reshape(128, 128)
    i2_2 = i_2.astype(jnp.int32).reshape(128, 128)
    r1_2 = r_1.reshape(128, 128)
    r2_2 = r_2.reshape(128, 128)
    total = _run(u2, i2, j2, i1_2, i2_2, W, H, H_1, H_2, r1_2, r2_2)
    return total / batch_size
